# baseline (speedup 1.0000x reference)
import functools

import jax
import jax.numpy as jnp
from jax import lax
from jax.experimental import pallas as pl
from jax.experimental.pallas import tpu as pltpu
from jax.experimental.pallas import tpu_sc as plsc

N = 10000
E = 320000
D = 128

NC = 2
NS = 16
NW = NC * NS
K = 80
EPW = E // NW
NCH = EPW // K
NBK = 5
CPB = NCH // NBK
NP = 10240
RPS = NP // NS


_MESH = plsc.VectorSubcoreMesh(core_axis_name="c", subcore_axis_name="s")


_RING = 4


def _make_sc_agg(with_deg):
    def body(xm, src4d, dst4d, z_big, ones_h, *refs):
        if with_deg:
            acc_out, deg_out, src_v, dst_v, rows_v, acc_s, *sems = refs
        else:
            acc_out, src_v, dst_v, rows_v, acc_s, *sems = refs
        gsems = sems[:_RING]
        ssems = sems[_RING:]
        c = lax.axis_index("c")
        s = lax.axis_index("s")
        wid = s * NC + c
        r0 = s * RPS

        def _zero():
            pltpu.sync_copy(z_big.at[pl.ds(r0, RPS)],
                            acc_s.at[pl.ds(r0, RPS)])

        def _g(j, b):
            pltpu.async_copy(xm.at[src_v.at[j]], rows_v.at[b], gsems[b])

        def _wg(j, b):
            pltpu.make_async_copy(xm.at[src_v.at[j]], rows_v.at[b],
                                  gsems[b]).wait()

        def _s(j, b):
            pltpu.async_copy(rows_v.at[b], acc_s.at[dst_v.at[j]],
                             ssems[b], add=True)

        def _ws(j, b):
            pltpu.make_async_copy(rows_v.at[b], acc_s.at[dst_v.at[j]],
                                  ssems[b]).wait()

        if with_deg:
            _zero()
            pltpu.sync_copy(ones_h, rows_v.at[0])
            plsc.subcore_barrier()

            def deg_blk(blk, carry):
                pltpu.sync_copy(dst4d.at[wid, blk], dst_v)
                for j in range(CPB):
                    pltpu.async_copy(rows_v.at[0], acc_s.at[dst_v.at[j]],
                                     ssems[0], add=True)
                for j in range(CPB):
                    pltpu.make_async_copy(rows_v.at[0],
                                          acc_s.at[dst_v.at[j]],
                                          ssems[0]).wait()
                return carry

            lax.fori_loop(0, NBK, deg_blk, 0)
            plsc.subcore_barrier()
            pltpu.sync_copy(acc_s.at[pl.ds(r0, RPS)],
                            deg_out.at[c, pl.ds(r0, RPS)])

        _zero()
        plsc.subcore_barrier()

        def blk_body(blk, carry):
            pltpu.sync_copy(src4d.at[wid, blk], src_v)
            pltpu.sync_copy(dst4d.at[wid, blk], dst_v)

            for j in range(_RING - 1):
                _g(j, j)
            for j in range(CPB):
                b = j % _RING
                _wg(j, b)
                _s(j, b)
                nj = j + _RING - 1
                if nj < CPB:
                    bb = nj % _RING
                    if j >= 1:
                        _ws(j - 1, bb)
                    _g(nj, bb)
            for j in range(max(0, CPB - _RING + 1), CPB):
                _ws(j, j % _RING)
            return carry

        lax.fori_loop(0, NBK, blk_body, 0)
        plsc.subcore_barrier()
        pltpu.sync_copy(acc_s.at[pl.ds(r0, RPS)],
                        acc_out.at[c, pl.ds(r0, RPS)])

    out_type = [jax.ShapeDtypeStruct((NC, NP, D), jnp.float32)]
    if with_deg:
        out_type.append(jax.ShapeDtypeStruct((NC, NP, D), jnp.float32))
    return pl.kernel(
        body,
        out_type=out_type,
        mesh=_MESH,
        scratch_types=[
            pltpu.VMEM((CPB, K), jnp.int32),
            pltpu.VMEM((CPB, K), jnp.int32),
            pltpu.VMEM((_RING, K, D), jnp.float32),
            pltpu.VMEM_SHARED((NP, D), jnp.float32),
        ] + [pltpu.SemaphoreType.DMA] * (2 * _RING))


_sc_agg_deg = _make_sc_agg(True)
_sc_agg = _make_sc_agg(False)


def _tc_pre_body(x_ref, wn_ref, ws_ref, b_ref, xm_ref, xs_ref):
    xv = x_ref[...]
    xm_ref[...] = jnp.dot(xv, wn_ref[...], preferred_element_type=jnp.float32)
    xs_ref[...] = (jnp.dot(xv, ws_ref[...], preferred_element_type=jnp.float32)
                   + b_ref[...])


def _bn(h, g, b):
    mu = jnp.mean(h, axis=0, keepdims=True)
    var = jnp.mean((h - mu) ** 2, axis=0, keepdims=True)
    return (h - mu) * lax.rsqrt(var + 1e-5) * g + b


def _tc_mid_body(xs_ref, acc_ref, dacc_ref, wn2_ref, ws2_ref, b2_ref,
                 wsk_ref, bsk_ref, g1_ref, be1_ref, gsk_ref, besk_ref,
                 h1m_ref, h1s_ref, skip_ref):
    da = dacc_ref[...]
    deg = jnp.maximum(da[0, :N, 0:1] + da[1, :N, 0:1], 1.0)
    a = acc_ref[...]
    h = xs_ref[...] + (a[0, :N] + a[1, :N]) / deg
    h1 = jnp.maximum(_bn(h, g1_ref[...], be1_ref[...]), 0.0)
    h1m_ref[...] = jnp.dot(h1, wn2_ref[...],
                           preferred_element_type=jnp.float32)
    h1s_ref[...] = (jnp.dot(h1, ws2_ref[...],
                            preferred_element_type=jnp.float32) + b2_ref[...])
    sk = (jnp.dot(h1, wsk_ref[...], preferred_element_type=jnp.float32)
          + bsk_ref[...])
    skip_ref[...] = _bn(sk, gsk_ref[...], besk_ref[...])


def _tc_post_body(h1s_ref, acc2_ref, dacc_ref, skip_ref, g2_ref, be2_ref,
                  out_ref):
    da = dacc_ref[...]
    deg = jnp.maximum(da[0, :N, 0:1] + da[1, :N, 0:1], 1.0)
    a = acc2_ref[...]
    h = h1s_ref[...] + (a[0, :N] + a[1, :N]) / deg
    h2 = _bn(h, g2_ref[...], be2_ref[...])
    out_ref[...] = jnp.maximum(h2 + skip_ref[...], 0.0)


_f32 = jnp.float32


def kernel(x, edge_index, W_self1, W_neigh1, b1, gamma1, beta1,
           W_self2, W_neigh2, b2, gamma2, beta2,
           W_skip, b_skip, gamma_skip, beta_skip):
    src4d = edge_index[0].reshape(NW, NBK, CPB, K)
    dst4d = edge_index[1].reshape(NW, NBK, CPB, K)
    z_big = jnp.zeros((NP, D), _f32)
    ones_h = jnp.ones((K, D), _f32)

    xm1, xs1 = pl.pallas_call(
        _tc_pre_body,
        out_shape=[jax.ShapeDtypeStruct((N, D), _f32)] * 2,
    )(x, W_neigh1, W_self1, b1.reshape(1, D))

    acc1, dacc = _sc_agg_deg(xm1, src4d, dst4d, z_big, ones_h)

    h1m, h1s, skipbn = pl.pallas_call(
        _tc_mid_body,
        out_shape=[jax.ShapeDtypeStruct((N, D), _f32)] * 3,
    )(xs1, acc1, dacc, W_neigh2, W_self2, b2.reshape(1, D),
      W_skip, b_skip.reshape(1, D), gamma1.reshape(1, D), beta1.reshape(1, D),
      gamma_skip.reshape(1, D), beta_skip.reshape(1, D))

    (acc2,) = _sc_agg(h1m, src4d, dst4d, z_big, ones_h)

    out = pl.pallas_call(
        _tc_post_body,
        out_shape=jax.ShapeDtypeStruct((N, D), _f32),
    )(h1s, acc2, dacc, skipbn, gamma2.reshape(1, D), beta2.reshape(1, D))
    return out


# speedup vs baseline: 1.0250x; 1.0250x over previous
import functools

import jax
import jax.numpy as jnp
from jax import lax
from jax.experimental import pallas as pl
from jax.experimental.pallas import tpu as pltpu
from jax.experimental.pallas import tpu_sc as plsc

N = 10000
E = 320000
D = 128

NC = 2
NS = 16
NW = NC * NS
K = 80
EPW = E // NW
NCH = EPW // K
NBK = 5
CPB = NCH // NBK
NP = 10240
RPS = NP // NS


_MESH = plsc.VectorSubcoreMesh(core_axis_name="c", subcore_axis_name="s")


_RING = 3


def _make_sc_agg(with_deg):
    def body(xm, src4d, dst4d, z_big, ones_h, *refs):
        if with_deg:
            acc_out, deg_out, src_v, dst_v, rows_v, acc_s, *sems = refs
        else:
            acc_out, src_v, dst_v, rows_v, acc_s, *sems = refs
        gsems = sems[:_RING]
        ssems = sems[_RING:]
        c = lax.axis_index("c")
        s = lax.axis_index("s")
        wid = s * NC + c
        r0 = s * RPS

        def _zero():
            pltpu.sync_copy(z_big.at[pl.ds(r0, RPS)],
                            acc_s.at[pl.ds(r0, RPS)])

        def _g(j, b):
            pltpu.async_copy(xm.at[src_v.at[j]], rows_v.at[b], gsems[b])

        def _wg(j, b):
            pltpu.make_async_copy(xm.at[src_v.at[j]], rows_v.at[b],
                                  gsems[b]).wait()

        def _s(j, b):
            pltpu.async_copy(rows_v.at[b], acc_s.at[dst_v.at[j]],
                             ssems[b], add=True)

        def _ws(j, b):
            pltpu.make_async_copy(rows_v.at[b], acc_s.at[dst_v.at[j]],
                                  ssems[b]).wait()

        if with_deg:
            _zero()
            pltpu.sync_copy(ones_h, rows_v.at[0])
            plsc.subcore_barrier()

            def deg_blk(blk, carry):
                pltpu.sync_copy(dst4d.at[wid, blk], dst_v)
                for j in range(CPB):
                    pltpu.async_copy(rows_v.at[0], acc_s.at[dst_v.at[j]],
                                     ssems[0], add=True)
                for j in range(CPB):
                    pltpu.make_async_copy(rows_v.at[0],
                                          acc_s.at[dst_v.at[j]],
                                          ssems[0]).wait()
                return carry

            lax.fori_loop(0, NBK, deg_blk, 0)
            plsc.subcore_barrier()
            pltpu.sync_copy(acc_s.at[pl.ds(r0, RPS)],
                            deg_out.at[c, pl.ds(r0, RPS)])

        _zero()
        plsc.subcore_barrier()

        def blk_body(blk, carry):
            pltpu.sync_copy(src4d.at[wid, blk], src_v)
            pltpu.sync_copy(dst4d.at[wid, blk], dst_v)

            for j in range(_RING - 1):
                _g(j, j)
            for j in range(CPB):
                b = j % _RING
                _wg(j, b)
                _s(j, b)
                nj = j + _RING - 1
                if nj < CPB:
                    bb = nj % _RING
                    if j >= 1:
                        _ws(j - 1, bb)
                    _g(nj, bb)
            for j in range(max(0, CPB - _RING + 1), CPB):
                _ws(j, j % _RING)
            return carry

        lax.fori_loop(0, NBK, blk_body, 0)
        plsc.subcore_barrier()
        pltpu.sync_copy(acc_s.at[pl.ds(r0, RPS)],
                        acc_out.at[c, pl.ds(r0, RPS)])

    out_type = [jax.ShapeDtypeStruct((NC, NP, D), jnp.float32)]
    if with_deg:
        out_type.append(jax.ShapeDtypeStruct((NC, NP, D), jnp.float32))
    return pl.kernel(
        body,
        out_type=out_type,
        mesh=_MESH,
        scratch_types=[
            pltpu.VMEM((CPB, K), jnp.int32),
            pltpu.VMEM((CPB, K), jnp.int32),
            pltpu.VMEM((_RING, K, D), jnp.float32),
            pltpu.VMEM_SHARED((NP, D), jnp.float32),
        ] + [pltpu.SemaphoreType.DMA] * (2 * _RING))


_sc_agg_deg = _make_sc_agg(True)
_sc_agg = _make_sc_agg(False)


def _tc_pre_body(x_ref, wn_ref, ws_ref, b_ref, xm_ref, xs_ref):
    xv = x_ref[...]
    xm_ref[...] = jnp.dot(xv, wn_ref[...], preferred_element_type=jnp.float32)
    xs_ref[...] = (jnp.dot(xv, ws_ref[...], preferred_element_type=jnp.float32)
                   + b_ref[...])


def _bn(h, g, b):
    mu = jnp.mean(h, axis=0, keepdims=True)
    var = jnp.mean((h - mu) ** 2, axis=0, keepdims=True)
    return (h - mu) * lax.rsqrt(var + 1e-5) * g + b


def _tc_mid_body(xs_ref, acc_ref, dacc_ref, wn2_ref, ws2_ref, b2_ref,
                 wsk_ref, bsk_ref, g1_ref, be1_ref, gsk_ref, besk_ref,
                 h1m_ref, h1s_ref, skip_ref):
    da = dacc_ref[...]
    deg = jnp.maximum(da[0, :N, 0:1] + da[1, :N, 0:1], 1.0)
    a = acc_ref[...]
    h = xs_ref[...] + (a[0, :N] + a[1, :N]) / deg
    h1 = jnp.maximum(_bn(h, g1_ref[...], be1_ref[...]), 0.0)
    h1m_ref[...] = jnp.dot(h1, wn2_ref[...],
                           preferred_element_type=jnp.float32)
    h1s_ref[...] = (jnp.dot(h1, ws2_ref[...],
                            preferred_element_type=jnp.float32) + b2_ref[...])
    sk = (jnp.dot(h1, wsk_ref[...], preferred_element_type=jnp.float32)
          + bsk_ref[...])
    skip_ref[...] = _bn(sk, gsk_ref[...], besk_ref[...])


def _tc_post_body(h1s_ref, acc2_ref, dacc_ref, skip_ref, g2_ref, be2_ref,
                  out_ref):
    da = dacc_ref[...]
    deg = jnp.maximum(da[0, :N, 0:1] + da[1, :N, 0:1], 1.0)
    a = acc2_ref[...]
    h = h1s_ref[...] + (a[0, :N] + a[1, :N]) / deg
    h2 = _bn(h, g2_ref[...], be2_ref[...])
    out_ref[...] = jnp.maximum(h2 + skip_ref[...], 0.0)


_f32 = jnp.float32


def kernel(x, edge_index, W_self1, W_neigh1, b1, gamma1, beta1,
           W_self2, W_neigh2, b2, gamma2, beta2,
           W_skip, b_skip, gamma_skip, beta_skip):
    src4d = edge_index[0].reshape(NW, NBK, CPB, K)
    dst4d = edge_index[1].reshape(NW, NBK, CPB, K)
    z_big = jnp.zeros((NP, D), _f32)
    ones_h = jnp.ones((K, D), _f32)

    xm1, xs1 = pl.pallas_call(
        _tc_pre_body,
        out_shape=[jax.ShapeDtypeStruct((N, D), _f32)] * 2,
    )(x, W_neigh1, W_self1, b1.reshape(1, D))

    acc1, dacc = _sc_agg_deg(xm1, src4d, dst4d, z_big, ones_h)

    h1m, h1s, skipbn = pl.pallas_call(
        _tc_mid_body,
        out_shape=[jax.ShapeDtypeStruct((N, D), _f32)] * 3,
    )(xs1, acc1, dacc, W_neigh2, W_self2, b2.reshape(1, D),
      W_skip, b_skip.reshape(1, D), gamma1.reshape(1, D), beta1.reshape(1, D),
      gamma_skip.reshape(1, D), beta_skip.reshape(1, D))

    (acc2,) = _sc_agg(h1m, src4d, dst4d, z_big, ones_h)

    out = pl.pallas_call(
        _tc_post_body,
        out_shape=jax.ShapeDtypeStruct((N, D), _f32),
    )(h1s, acc2, dacc, skipbn, gamma2.reshape(1, D), beta2.reshape(1, D))
    return out


# speedup vs baseline: 1.0313x; 1.0061x over previous
import functools

import jax
import jax.numpy as jnp
from jax import lax
from jax.experimental import pallas as pl
from jax.experimental.pallas import tpu as pltpu
from jax.experimental.pallas import tpu_sc as plsc

N = 10000
E = 320000
D = 128

NC = 2
NS = 16
NW = NC * NS
K = 80
EPW = E // NW
NCH = EPW // K
NBK = 5
CPB = NCH // NBK
NP = 10240
RPS = NP // NS


_MESH = plsc.VectorSubcoreMesh(core_axis_name="c", subcore_axis_name="s")


_RING = 3


def _make_sc_agg(with_deg):
    def body(xm, src4d, dst4d, z_big, ones_h, *refs):
        if with_deg:
            acc_out, deg_out, src_v, dst_v, rows_v, acc_s, *sems = refs
        else:
            acc_out, src_v, dst_v, rows_v, acc_s, *sems = refs
        gsems = sems[:_RING]
        ssems = sems[_RING:]
        c = lax.axis_index("c")
        s = lax.axis_index("s")
        wid = s * NC + c
        r0 = s * RPS

        def _zero():
            pltpu.sync_copy(z_big.at[pl.ds(r0, RPS)],
                            acc_s.at[pl.ds(r0, RPS)])

        def _g(j, b):
            pltpu.async_copy(xm.at[src_v.at[j]], rows_v.at[b], gsems[b])

        def _wg(j, b):
            pltpu.make_async_copy(xm.at[src_v.at[j]], rows_v.at[b],
                                  gsems[b]).wait()

        def _s(j, b):
            pltpu.async_copy(rows_v.at[b], acc_s.at[dst_v.at[j]],
                             ssems[b], add=True)

        def _ws(j, b):
            pltpu.make_async_copy(rows_v.at[b], acc_s.at[dst_v.at[j]],
                                  ssems[b]).wait()

        if with_deg:
            _zero()
            pltpu.sync_copy(ones_h, rows_v.at[0])
            plsc.subcore_barrier()

            def deg_blk(blk, carry):
                pltpu.sync_copy(dst4d.at[wid, blk], dst_v)
                for j in range(CPB):
                    pltpu.async_copy(rows_v.at[0], acc_s.at[dst_v.at[j]],
                                     ssems[0], add=True)
                for j in range(CPB):
                    pltpu.make_async_copy(rows_v.at[0],
                                          acc_s.at[dst_v.at[j]],
                                          ssems[0]).wait()
                return carry

            lax.fori_loop(0, NBK, deg_blk, 0)
            plsc.subcore_barrier()
            pltpu.sync_copy(acc_s.at[pl.ds(r0, RPS)],
                            deg_out.at[c, pl.ds(r0, RPS)])

        _zero()
        plsc.subcore_barrier()

        def blk_body(blk, carry):
            pltpu.sync_copy(src4d.at[wid, blk], src_v)
            pltpu.sync_copy(dst4d.at[wid, blk], dst_v)

            for j in range(_RING - 1):
                _g(j, j)
            for j in range(CPB):
                b = j % _RING
                _wg(j, b)
                _s(j, b)
                nj = j + _RING - 1
                if nj < CPB:
                    bb = nj % _RING
                    if j >= 1:
                        _ws(j - 1, bb)
                    _g(nj, bb)
            for j in range(max(0, CPB - _RING), CPB):
                _ws(j, j % _RING)
            return carry

        lax.fori_loop(0, NBK, blk_body, 0)
        plsc.subcore_barrier()
        pltpu.sync_copy(acc_s.at[pl.ds(r0, RPS)],
                        acc_out.at[c, pl.ds(r0, RPS)])

    out_type = [jax.ShapeDtypeStruct((NC, NP, D), jnp.float32)]
    if with_deg:
        out_type.append(jax.ShapeDtypeStruct((NC, NP, D), jnp.float32))
    return pl.kernel(
        body,
        out_type=out_type,
        mesh=_MESH,
        scratch_types=[
            pltpu.VMEM((CPB, K), jnp.int32),
            pltpu.VMEM((CPB, K), jnp.int32),
            pltpu.VMEM((_RING, K, D), jnp.float32),
            pltpu.VMEM_SHARED((NP, D), jnp.float32),
        ] + [pltpu.SemaphoreType.DMA] * (2 * _RING))


_sc_agg_deg = _make_sc_agg(True)
_sc_agg = _make_sc_agg(False)


def _tc_pre_body(x_ref, wn_ref, ws_ref, b_ref, xm_ref, xs_ref):
    xv = x_ref[...]
    xm_ref[...] = jnp.dot(xv, wn_ref[...], preferred_element_type=jnp.float32)
    xs_ref[...] = (jnp.dot(xv, ws_ref[...], preferred_element_type=jnp.float32)
                   + b_ref[...])


def _bn(h, g, b):
    mu = jnp.mean(h, axis=0, keepdims=True)
    var = jnp.mean((h - mu) ** 2, axis=0, keepdims=True)
    return (h - mu) * lax.rsqrt(var + 1e-5) * g + b


def _tc_mid_body(xs_ref, acc_ref, dacc_ref, wn2_ref, ws2_ref, b2_ref,
                 wsk_ref, bsk_ref, g1_ref, be1_ref, gsk_ref, besk_ref,
                 h1m_ref, h1s_ref, skip_ref):
    da = dacc_ref[...]
    deg = jnp.maximum(da[0, :N, 0:1] + da[1, :N, 0:1], 1.0)
    a = acc_ref[...]
    h = xs_ref[...] + (a[0, :N] + a[1, :N]) / deg
    h1 = jnp.maximum(_bn(h, g1_ref[...], be1_ref[...]), 0.0)
    h1m_ref[...] = jnp.dot(h1, wn2_ref[...],
                           preferred_element_type=jnp.float32)
    h1s_ref[...] = (jnp.dot(h1, ws2_ref[...],
                            preferred_element_type=jnp.float32) + b2_ref[...])
    sk = (jnp.dot(h1, wsk_ref[...], preferred_element_type=jnp.float32)
          + bsk_ref[...])
    skip_ref[...] = _bn(sk, gsk_ref[...], besk_ref[...])


def _tc_post_body(h1s_ref, acc2_ref, dacc_ref, skip_ref, g2_ref, be2_ref,
                  out_ref):
    da = dacc_ref[...]
    deg = jnp.maximum(da[0, :N, 0:1] + da[1, :N, 0:1], 1.0)
    a = acc2_ref[...]
    h = h1s_ref[...] + (a[0, :N] + a[1, :N]) / deg
    h2 = _bn(h, g2_ref[...], be2_ref[...])
    out_ref[...] = jnp.maximum(h2 + skip_ref[...], 0.0)


_f32 = jnp.float32


def kernel(x, edge_index, W_self1, W_neigh1, b1, gamma1, beta1,
           W_self2, W_neigh2, b2, gamma2, beta2,
           W_skip, b_skip, gamma_skip, beta_skip):
    src4d = edge_index[0].reshape(NW, NBK, CPB, K)
    dst4d = edge_index[1].reshape(NW, NBK, CPB, K)
    z_big = jnp.zeros((NP, D), _f32)
    ones_h = jnp.ones((K, D), _f32)

    xm1, xs1 = pl.pallas_call(
        _tc_pre_body,
        out_shape=[jax.ShapeDtypeStruct((N, D), _f32)] * 2,
    )(x, W_neigh1, W_self1, b1.reshape(1, D))

    acc1, dacc = _sc_agg_deg(xm1, src4d, dst4d, z_big, ones_h)

    h1m, h1s, skipbn = pl.pallas_call(
        _tc_mid_body,
        out_shape=[jax.ShapeDtypeStruct((N, D), _f32)] * 3,
    )(xs1, acc1, dacc, W_neigh2, W_self2, b2.reshape(1, D),
      W_skip, b_skip.reshape(1, D), gamma1.reshape(1, D), beta1.reshape(1, D),
      gamma_skip.reshape(1, D), beta_skip.reshape(1, D))

    (acc2,) = _sc_agg(h1m, src4d, dst4d, z_big, ones_h)

    out = pl.pallas_call(
        _tc_post_body,
        out_shape=jax.ShapeDtypeStruct((N, D), _f32),
    )(h1s, acc2, dacc, skipbn, gamma2.reshape(1, D), beta2.reshape(1, D))
    return out


# speedup vs baseline: 1.0326x; 1.0013x over previous
import functools

import jax
import jax.numpy as jnp
from jax import lax
from jax.experimental import pallas as pl
from jax.experimental.pallas import tpu as pltpu
from jax.experimental.pallas import tpu_sc as plsc

N = 10000
E = 320000
D = 128

NC = 2
NS = 16
NW = NC * NS
K = 80
EPW = E // NW
NCH = EPW // K
NBK = 5
CPB = NCH // NBK
NP = 10240
RPS = NP // NS


_MESH = plsc.VectorSubcoreMesh(core_axis_name="c", subcore_axis_name="s")


_RING = 3


def _make_sc_agg(with_deg):
    def body(xm, src4d, dst4d, z_big, ones_h, *refs):
        if with_deg:
            acc_out, deg_out, src_v, dst_v, rows_v, acc_s, *sems = refs
        else:
            acc_out, src_v, dst_v, rows_v, acc_s, *sems = refs
        gsems = sems[:_RING]
        ssems = sems[_RING:]
        c = lax.axis_index("c")
        s = lax.axis_index("s")
        wid = s * NC + c
        r0 = s * RPS

        def _zero():
            pltpu.sync_copy(z_big.at[pl.ds(r0, RPS)],
                            acc_s.at[pl.ds(r0, RPS)])

        def _g(j, b):
            pltpu.async_copy(xm.at[src_v.at[j]], rows_v.at[b], gsems[b])

        def _wg(j, b):
            pltpu.make_async_copy(xm.at[src_v.at[j]], rows_v.at[b],
                                  gsems[b]).wait()

        def _s(j, b):
            pltpu.async_copy(rows_v.at[b], acc_s.at[dst_v.at[j]],
                             ssems[b], add=True)

        def _ws(j, b):
            pltpu.make_async_copy(rows_v.at[b], acc_s.at[dst_v.at[j]],
                                  ssems[b]).wait()

        if with_deg:
            _zero()
            pltpu.sync_copy(ones_h, rows_v.at[0])
            plsc.subcore_barrier()

            def deg_blk(blk, carry):
                pltpu.sync_copy(dst4d.at[wid, blk], dst_v)
                for j in range(CPB):
                    pltpu.async_copy(rows_v.at[0], acc_s.at[dst_v.at[j]],
                                     ssems[0], add=True)
                for j in range(CPB):
                    pltpu.make_async_copy(rows_v.at[0],
                                          acc_s.at[dst_v.at[j]],
                                          ssems[0]).wait()
                return carry

            lax.fori_loop(0, NBK, deg_blk, 0)
            plsc.subcore_barrier()
            pltpu.sync_copy(acc_s.at[pl.ds(r0, RPS)],
                            deg_out.at[c, pl.ds(r0, RPS)])

        _zero()
        plsc.subcore_barrier()

        def blk_body(blk, carry):
            pltpu.sync_copy(src4d.at[wid, blk], src_v)
            pltpu.sync_copy(dst4d.at[wid, blk], dst_v)

            for j in range(_RING - 1):
                _g(j, j)
            for j in range(CPB):
                b = j % _RING
                _wg(j, b)
                _s(j, b)
                nj = j + _RING - 1
                if nj < CPB:
                    bb = nj % _RING
                    if j >= 1:
                        _ws(j - 1, bb)
                    _g(nj, bb)
            for j in range(max(0, CPB - _RING), CPB):
                _ws(j, j % _RING)
            return carry

        lax.fori_loop(0, NBK, blk_body, 0)
        plsc.subcore_barrier()
        pltpu.sync_copy(acc_s.at[pl.ds(r0, RPS)],
                        acc_out.at[c, pl.ds(r0, RPS)])

    out_type = [jax.ShapeDtypeStruct((NC, NP, D), jnp.float32)]
    if with_deg:
        out_type.append(jax.ShapeDtypeStruct((NC, NP, D), jnp.float32))
    return pl.kernel(
        body,
        out_type=out_type,
        mesh=_MESH,
        scratch_types=[
            pltpu.VMEM((CPB, K), jnp.int32),
            pltpu.VMEM((CPB, K), jnp.int32),
            pltpu.VMEM((_RING, K, D), jnp.float32),
            pltpu.VMEM_SHARED((NP, D), jnp.float32),
        ] + [pltpu.SemaphoreType.DMA] * (2 * _RING))


_sc_agg_deg = _make_sc_agg(True)
_sc_agg = _make_sc_agg(False)


def _tc_pre_body(x_ref, wn_ref, ws_ref, b_ref, xm_ref, xs_ref):
    xv = x_ref[...]
    xm_ref[...] = jnp.dot(xv, wn_ref[...], preferred_element_type=jnp.float32)
    xs_ref[...] = (jnp.dot(xv, ws_ref[...], preferred_element_type=jnp.float32)
                   + b_ref[...])


def _bn(h, g, b):
    mu = jnp.mean(h, axis=0, keepdims=True)
    var = jnp.mean((h - mu) ** 2, axis=0, keepdims=True)
    return (h - mu) * lax.rsqrt(var + 1e-5) * g + b


def _tc_mid_body(xs_ref, acc_ref, dacc_ref, wn2_ref, ws2_ref, b2_ref,
                 wsk_ref, bsk_ref, g1_ref, be1_ref, gsk_ref, besk_ref,
                 h1m_ref, h1s_ref, skip_ref):
    da = dacc_ref[...]
    deg = jnp.maximum(da[0, :N] + da[1, :N], 1.0)
    a = acc_ref[...]
    h = xs_ref[...] + (a[0, :N] + a[1, :N]) / deg
    h1 = jnp.maximum(_bn(h, g1_ref[...], be1_ref[...]), 0.0)
    h1m_ref[...] = jnp.dot(h1, wn2_ref[...],
                           preferred_element_type=jnp.float32)
    h1s_ref[...] = (jnp.dot(h1, ws2_ref[...],
                            preferred_element_type=jnp.float32) + b2_ref[...])
    sk = (jnp.dot(h1, wsk_ref[...], preferred_element_type=jnp.float32)
          + bsk_ref[...])
    skip_ref[...] = _bn(sk, gsk_ref[...], besk_ref[...])


def _tc_post_body(h1s_ref, acc2_ref, dacc_ref, skip_ref, g2_ref, be2_ref,
                  out_ref):
    da = dacc_ref[...]
    deg = jnp.maximum(da[0, :N] + da[1, :N], 1.0)
    a = acc2_ref[...]
    h = h1s_ref[...] + (a[0, :N] + a[1, :N]) / deg
    h2 = _bn(h, g2_ref[...], be2_ref[...])
    out_ref[...] = jnp.maximum(h2 + skip_ref[...], 0.0)


_f32 = jnp.float32


def kernel(x, edge_index, W_self1, W_neigh1, b1, gamma1, beta1,
           W_self2, W_neigh2, b2, gamma2, beta2,
           W_skip, b_skip, gamma_skip, beta_skip):
    src4d = edge_index[0].reshape(NW, NBK, CPB, K)
    dst4d = edge_index[1].reshape(NW, NBK, CPB, K)
    z_big = jnp.zeros((NP, D), _f32)
    ones_h = jnp.ones((K, D), _f32)

    xm1, xs1 = pl.pallas_call(
        _tc_pre_body,
        out_shape=[jax.ShapeDtypeStruct((N, D), _f32)] * 2,
    )(x, W_neigh1, W_self1, b1.reshape(1, D))

    acc1, dacc = _sc_agg_deg(xm1, src4d, dst4d, z_big, ones_h)
    dacc = dacc[:, :, :1]

    h1m, h1s, skipbn = pl.pallas_call(
        _tc_mid_body,
        out_shape=[jax.ShapeDtypeStruct((N, D), _f32)] * 3,
    )(xs1, acc1, dacc, W_neigh2, W_self2, b2.reshape(1, D),
      W_skip, b_skip.reshape(1, D), gamma1.reshape(1, D), beta1.reshape(1, D),
      gamma_skip.reshape(1, D), beta_skip.reshape(1, D))

    (acc2,) = _sc_agg(h1m, src4d, dst4d, z_big, ones_h)

    out = pl.pallas_call(
        _tc_post_body,
        out_shape=jax.ShapeDtypeStruct((N, D), _f32),
    )(h1s, acc2, dacc, skipbn, gamma2.reshape(1, D), beta2.reshape(1, D))
    return out
